# Initial kernel scaffold; baseline (speedup 1.0000x reference)
import jax
import jax.numpy as jnp
from jax.experimental import pallas as pl


def kernel(feature, edge_index, W, b):
    raise NotImplementedError("write your pallas kernel here")


# speedup vs baseline: 6.2350x; 6.2350x over previous
import functools

import jax
import jax.numpy as jnp
from jax import lax
from jax.experimental import pallas as pl
from jax.experimental.pallas import tpu as pltpu
from jax.experimental.pallas import tpu_sc as plsc

_N = 10000
_D = 128
_E = 320000

_NC = 2
_NS = 16
_NW = _NC * _NS
_EPW = _E // _NW
_B = 80
_NB = _EPW // _B
_RCH = 16
_RPT = 624


def _sc_aggregate(feature, src, dst):
  mesh = plsc.VectorSubcoreMesh(core_axis_name="c", subcore_axis_name="s")

  @functools.partial(
      pl.kernel,
      out_type=(
          jax.ShapeDtypeStruct((_NC, _N, _D), jnp.float32),
          jax.ShapeDtypeStruct((_NC, _NS, _N), jnp.float32),
      ),
      mesh=mesh,
      compiler_params=pltpu.CompilerParams(needs_layout_passes=False),
      scratch_types=(
          pltpu.VMEM((_B,), jnp.int32),
          pltpu.VMEM((_B,), jnp.int32),
          pltpu.VMEM((_B, _D), jnp.float32),
          pltpu.VMEM((_N,), jnp.float32),
          pltpu.VMEM((_RCH, _D), jnp.float32),
          pltpu.VMEM_SHARED((_N, _D), jnp.float32),
          pltpu.SemaphoreType.DMA,
      ),
  )
  def k(feature_hbm, src_hbm, dst_hbm, agg_out, deg_out,
        src_v, dst_v, rows_v, hist_v, zrow_v, agg_sh, gsem):
    cid = lax.axis_index("c")
    sid = lax.axis_index("s")
    wid = sid * _NC + cid
    ebase = wid * _EPW
    rbase = sid * _RPT
    nch = jnp.where(sid == _NS - 1, 40, _RPT // _RCH)

    z16 = jnp.zeros((16,), jnp.float32)
    o16 = jnp.ones((16,), jnp.float32)

    @pl.loop(0, _RCH)
    def _zr(r):
      @pl.loop(0, _D // 16)
      def _zc(c):
        zrow_v[r, pl.ds(c * 16, 16)] = z16

    @pl.loop(0, _N // 16)
    def _zh(r):
      hist_v[pl.ds(r * 16, 16)] = z16

    @pl.loop(0, nch)
    def _zi(j):
      pltpu.sync_copy(zrow_v, agg_sh.at[pl.ds(rbase + j * _RCH, _RCH)])

    plsc.subcore_barrier()

    @pl.loop(0, _NB)
    def _eb(i):
      off = ebase + i * _B
      pltpu.sync_copy(src_hbm.at[pl.ds(off, _B)], src_v)
      pltpu.sync_copy(dst_hbm.at[pl.ds(off, _B)], dst_v)
      pltpu.async_copy(feature_hbm.at[src_v], rows_v, gsem).wait()
      pltpu.sync_copy(rows_v, agg_sh.at[dst_v], add=True)

      @pl.loop(0, _B // 16)
      def _dh(g):
        idx = dst_v[pl.ds(g * 16, 16)]
        plsc.addupdate_scatter(hist_v, [idx], o16)

    plsc.subcore_barrier()

    @pl.loop(0, nch)
    def _wo(j):
      r0 = rbase + j * _RCH
      pltpu.sync_copy(agg_sh.at[pl.ds(r0, _RCH)], zrow_v)
      pltpu.sync_copy(zrow_v, agg_out.at[cid, pl.ds(r0, _RCH)])

    pltpu.sync_copy(hist_v, deg_out.at[cid, sid])

  return k(feature, src, dst)


_R = 1000


def _tc_combine(feature, agg2, degT, W, b):
  def body(f_ref, a_ref, d_ref, w_ref, b_ref, o_ref):
    f = f_ref[...]
    a = a_ref[0] + a_ref[1]
    d = jnp.sum(d_ref[...], axis=1)
    deg = jnp.maximum(d, 1.0)[:, None]
    neigh = a / deg
    w = w_ref[...]
    acc = jnp.dot(f, w[:_D], preferred_element_type=jnp.float32)
    acc = acc + jnp.dot(neigh, w[_D:], preferred_element_type=jnp.float32)
    o_ref[...] = jnp.maximum(acc + b_ref[...], 0.0)

  return pl.pallas_call(
      body,
      grid=(_N // _R,),
      in_specs=[
          pl.BlockSpec((_R, _D), lambda i: (i, 0)),
          pl.BlockSpec((_NC, _R, _D), lambda i: (0, i, 0)),
          pl.BlockSpec((_R, _NC * _NS), lambda i: (i, 0)),
          pl.BlockSpec((2 * _D, _D), lambda i: (0, 0)),
          pl.BlockSpec((1, _D), lambda i: (0, 0)),
      ],
      out_specs=pl.BlockSpec((_R, _D), lambda i: (i, 0)),
      out_shape=jax.ShapeDtypeStruct((_N, _D), jnp.float32),
  )(feature, agg2, degT, W, b.reshape(1, _D))


def kernel(feature, edge_index, W, b):
  src = edge_index[0]
  dst = edge_index[1]
  agg2, deg2 = _sc_aggregate(feature, src, dst)
  degT = deg2.reshape(_NC * _NS, _N).T
  return _tc_combine(feature, agg2, degT, W, b)


# speedup vs baseline: 10.4196x; 1.6711x over previous
import functools

import jax
import jax.numpy as jnp
from jax import lax
from jax.experimental import pallas as pl
from jax.experimental.pallas import tpu as pltpu
from jax.experimental.pallas import tpu_sc as plsc

_N = 10000
_D = 128
_E = 320000

_NC = 2
_NS = 16
_NW = _NC * _NS
_EPW = _E // _NW
_B = 50
_NB = _EPW // _B
_CB = 40
_NCH = _NB // _CB
_CE = _CB * _B
_RCH = 16
_RPT = 624


def _sc_aggregate(feature, src, dst):
  mesh = plsc.VectorSubcoreMesh(core_axis_name="c", subcore_axis_name="s")

  @functools.partial(
      pl.kernel,
      out_type=(
          jax.ShapeDtypeStruct((_NC, _N, _D), jnp.float32),
          jax.ShapeDtypeStruct((_NC, _NS, _N), jnp.float32),
      ),
      mesh=mesh,
      compiler_params=pltpu.CompilerParams(needs_layout_passes=False),
      scratch_types=(
          pltpu.VMEM((_CB, _B), jnp.int32),
          pltpu.VMEM((_CB, _B), jnp.int32),
          pltpu.VMEM((_CE,), jnp.int32),
          pltpu.VMEM((2, _B, _D), jnp.float32),
          pltpu.VMEM((_N,), jnp.float32),
          pltpu.VMEM((_RCH, _D), jnp.float32),
          pltpu.VMEM_SHARED((_N, _D), jnp.float32),
          pltpu.SemaphoreType.DMA,
          pltpu.SemaphoreType.DMA,
      ),
  )
  def k(feature_hbm, src_hbm, dst_hbm, dstf_hbm, agg_out, deg_out,
        src_v, dst_v, dstf_v, rows_v, hist_v, zrow_v, agg_sh, gsem0, gsem1):
    cid = lax.axis_index("c")
    sid = lax.axis_index("s")
    wid = sid * _NC + cid
    rbase = sid * _RPT
    nch = jnp.where(sid == _NS - 1, 40, _RPT // _RCH)

    z16 = jnp.zeros((16,), jnp.float32)
    o16 = jnp.ones((16,), jnp.float32)

    @pl.loop(0, _RCH)
    def _zr(r):
      @pl.loop(0, _D // 16)
      def _zc(c):
        zrow_v[r, pl.ds(c * 16, 16)] = z16

    @pl.loop(0, _N // 16)
    def _zh(r):
      hist_v[pl.ds(r * 16, 16)] = z16

    @pl.loop(0, nch)
    def _zi(j):
      pltpu.sync_copy(zrow_v, agg_sh.at[pl.ds(rbase + j * _RCH, _RCH)])

    plsc.subcore_barrier()

    sems = (gsem0, gsem1)

    def _start_gather(i, buf):
      pltpu.async_copy(feature_hbm.at[src_v.at[i]], rows_v.at[buf], sems[buf])

    def _finish(i, buf):
      pltpu.make_async_copy(feature_hbm.at[src_v.at[i]], rows_v.at[buf],
                            sems[buf]).wait()
      pltpu.sync_copy(rows_v.at[buf], agg_sh.at[dst_v.at[i]], add=True)

    @pl.loop(0, _NCH)
    def _ch(c):
      pltpu.sync_copy(src_hbm.at[wid, pl.ds(c * _CB, _CB)], src_v)
      pltpu.sync_copy(dst_hbm.at[wid, pl.ds(c * _CB, _CB)], dst_v)
      pltpu.sync_copy(dstf_hbm.at[pl.ds(wid * _EPW + c * _CE, _CE)], dstf_v)
      _start_gather(0, 0)

      @pl.loop(0, _CB, step=2)
      def _eb(i):
        @pl.when(i + 1 < _CB)
        def _a():
          _start_gather(i + 1, 1)

        _finish(i, 0)

        @pl.when(i + 2 < _CB)
        def _b():
          _start_gather(i + 2, 0)

        @pl.loop(0, 6)
        def _dh(g):
          idx = dstf_v[pl.ds(i * 48 + g * 16, 16)]
          plsc.addupdate_scatter(hist_v, [idx], o16)

        @pl.when(i + 1 < _CB)
        def _c():
          _finish(i + 1, 1)

      @pl.loop(_CE // 16 - 5, _CE // 16)
      def _dt(g):
        idx = dstf_v[pl.ds(g * 16, 16)]
        plsc.addupdate_scatter(hist_v, [idx], o16)

    plsc.subcore_barrier()

    @pl.loop(0, nch)
    def _wo(j):
      r0 = rbase + j * _RCH
      pltpu.sync_copy(agg_sh.at[pl.ds(r0, _RCH)], zrow_v)
      pltpu.sync_copy(zrow_v, agg_out.at[cid, pl.ds(r0, _RCH)])

    pltpu.sync_copy(hist_v, deg_out.at[cid, sid])

  return k(feature, src.reshape(_NW, _NB, _B), dst.reshape(_NW, _NB, _B), dst)


_R = 1000


def _tc_combine(feature, agg2, degT, W, b):
  def body(f_ref, a_ref, d_ref, w_ref, b_ref, o_ref):
    f = f_ref[...]
    a = a_ref[0] + a_ref[1]
    d = jnp.sum(d_ref[...], axis=1)
    deg = jnp.maximum(d, 1.0)[:, None]
    neigh = a / deg
    w = w_ref[...]
    acc = jnp.dot(f, w[:_D], preferred_element_type=jnp.float32)
    acc = acc + jnp.dot(neigh, w[_D:], preferred_element_type=jnp.float32)
    o_ref[...] = jnp.maximum(acc + b_ref[...], 0.0)

  return pl.pallas_call(
      body,
      grid=(_N // _R,),
      in_specs=[
          pl.BlockSpec((_R, _D), lambda i: (i, 0)),
          pl.BlockSpec((_NC, _R, _D), lambda i: (0, i, 0)),
          pl.BlockSpec((_R, _NC * _NS), lambda i: (i, 0)),
          pl.BlockSpec((2 * _D, _D), lambda i: (0, 0)),
          pl.BlockSpec((1, _D), lambda i: (0, 0)),
      ],
      out_specs=pl.BlockSpec((_R, _D), lambda i: (i, 0)),
      out_shape=jax.ShapeDtypeStruct((_N, _D), jnp.float32),
  )(feature, agg2, degT, W, b.reshape(1, _D))


def kernel(feature, edge_index, W, b):
  src = edge_index[0]
  dst = edge_index[1]
  agg2, deg2 = _sc_aggregate(feature, src, dst)
  degT = deg2.reshape(_NC * _NS, _N).T
  return _tc_combine(feature, agg2, degT, W, b)


# speedup vs baseline: 11.1614x; 1.0712x over previous
import functools

import jax
import jax.numpy as jnp
from jax import lax
from jax.experimental import pallas as pl
from jax.experimental.pallas import tpu as pltpu
from jax.experimental.pallas import tpu_sc as plsc

_N = 10000
_D = 128
_E = 320000

_NC = 2
_NS = 16
_NW = _NC * _NS
_EPW = _E // _NW
_B = 50
_NB = _EPW // _B
_CB = 40
_NCH = _NB // _CB
_CE = _CB * _B
_RCH = 16
_RPT = 624


def _sc_aggregate(feature, src, dst):
  mesh = plsc.VectorSubcoreMesh(core_axis_name="c", subcore_axis_name="s")

  @functools.partial(
      pl.kernel,
      out_type=(
          jax.ShapeDtypeStruct((_NC, _N, _D), jnp.float32),
          jax.ShapeDtypeStruct((_NC, _NS, _N), jnp.float32),
      ),
      mesh=mesh,
      compiler_params=pltpu.CompilerParams(needs_layout_passes=False),
      scratch_types=(
          pltpu.VMEM((_CB, _B), jnp.int32),
          pltpu.VMEM((_CB, _B), jnp.int32),
          pltpu.VMEM((3, _B, _D), jnp.float32),
          pltpu.VMEM((_N,), jnp.float32),
          pltpu.VMEM((_RCH, _D), jnp.float32),
          pltpu.VMEM_SHARED((_N, _D), jnp.float32),
          pltpu.SemaphoreType.DMA,
          pltpu.SemaphoreType.DMA,
          pltpu.SemaphoreType.DMA,
          pltpu.SemaphoreType.DMA,
          pltpu.SemaphoreType.DMA,
          pltpu.SemaphoreType.DMA,
      ),
  )
  def k(feature_hbm, src_hbm, dst_hbm, agg_out, deg_out,
        src_v, dst_v, rows_v, hist_v, zrow_v, agg_sh,
        gsem0, gsem1, gsem2, ssem0, ssem1, ssem2):
    cid = lax.axis_index("c")
    sid = lax.axis_index("s")
    wid = sid * _NC + cid
    rbase = sid * _RPT
    nch = jnp.where(sid == _NS - 1, 40, _RPT // _RCH)

    z16 = jnp.zeros((16,), jnp.float32)
    o16 = jnp.ones((16,), jnp.float32)

    @pl.loop(0, _RCH)
    def _zr(r):
      @pl.loop(0, _D // 16)
      def _zc(c):
        zrow_v[r, pl.ds(c * 16, 16)] = z16

    @pl.loop(0, _N // 16)
    def _zh(r):
      hist_v[pl.ds(r * 16, 16)] = z16

    @pl.loop(0, nch)
    def _zi(j):
      pltpu.sync_copy(zrow_v, agg_sh.at[pl.ds(rbase + j * _RCH, _RCH)])

    plsc.subcore_barrier()

    gsems = (gsem0, gsem1, gsem2)
    ssems = (ssem0, ssem1, ssem2)
    tailmask = lax.iota(jnp.int32, 16) >= (16 - _B % 16)

    def _start_gather(i, buf):
      pltpu.async_copy(feature_hbm.at[src_v.at[i]], rows_v.at[buf], gsems[buf])

    def _wait_gather(i, buf):
      pltpu.make_async_copy(feature_hbm.at[src_v.at[i]], rows_v.at[buf],
                            gsems[buf]).wait()

    def _start_scatter(i, buf):
      pltpu.async_copy(rows_v.at[buf], agg_sh.at[dst_v.at[i]], ssems[buf],
                       add=True)

    def _wait_scatter(i, buf):
      pltpu.make_async_copy(rows_v.at[buf], agg_sh.at[dst_v.at[i]],
                            ssems[buf]).wait()

    def _hist(i):
      @pl.loop(0, _B // 16)
      def _dh(g):
        plsc.addupdate_scatter(hist_v, [dst_v[i, pl.ds(g * 16, 16)]], o16)

      plsc.addupdate_scatter(hist_v, [dst_v[i, pl.ds(_B - 16, 16)]], o16,
                             mask=tailmask)

    @pl.loop(0, _NCH)
    def _ch(c):
      pltpu.sync_copy(src_hbm.at[wid, pl.ds(c * _CB, _CB)], src_v)
      pltpu.sync_copy(dst_hbm.at[wid, pl.ds(c * _CB, _CB)], dst_v)
      for j in range(3):
        _start_gather(j, j)

      @pl.loop(0, _CB, step=3)
      def _eb(i):
        for j in range(3):
          k = i + j

          @pl.when(k < _CB)
          def _one():
            _wait_gather(k, j)
            _start_scatter(k, j)
            _hist(k)

        for j in range(3):
          k = i + j

          @pl.when(k + 3 < _CB)
          def _reuse():
            _wait_scatter(k, j)
            _start_gather(k + 3, j)

      for k in range(_CB - 3, _CB):
        _wait_scatter(k, k % 3)

    plsc.subcore_barrier()

    @pl.loop(0, nch)
    def _wo(j):
      r0 = rbase + j * _RCH
      pltpu.sync_copy(agg_sh.at[pl.ds(r0, _RCH)], zrow_v)
      pltpu.sync_copy(zrow_v, agg_out.at[cid, pl.ds(r0, _RCH)])

    pltpu.sync_copy(hist_v, deg_out.at[cid, sid])

  return k(feature, src.reshape(_NW, _NB, _B), dst.reshape(_NW, _NB, _B))


_R = 1000


def _tc_combine(feature, agg2, degT, W, b):
  def body(f_ref, a_ref, d_ref, w_ref, b_ref, o_ref):
    f = f_ref[...]
    a = a_ref[0] + a_ref[1]
    d = jnp.sum(d_ref[...], axis=1)
    deg = jnp.maximum(d, 1.0)[:, None]
    neigh = a / deg
    w = w_ref[...]
    acc = jnp.dot(f, w[:_D], preferred_element_type=jnp.float32)
    acc = acc + jnp.dot(neigh, w[_D:], preferred_element_type=jnp.float32)
    o_ref[...] = jnp.maximum(acc + b_ref[...], 0.0)

  return pl.pallas_call(
      body,
      grid=(_N // _R,),
      in_specs=[
          pl.BlockSpec((_R, _D), lambda i: (i, 0)),
          pl.BlockSpec((_NC, _R, _D), lambda i: (0, i, 0)),
          pl.BlockSpec((_R, _NC * _NS), lambda i: (i, 0)),
          pl.BlockSpec((2 * _D, _D), lambda i: (0, 0)),
          pl.BlockSpec((1, _D), lambda i: (0, 0)),
      ],
      out_specs=pl.BlockSpec((_R, _D), lambda i: (i, 0)),
      out_shape=jax.ShapeDtypeStruct((_N, _D), jnp.float32),
  )(feature, agg2, degT, W, b.reshape(1, _D))


def kernel(feature, edge_index, W, b):
  src = edge_index[0]
  dst = edge_index[1]
  agg2, deg2 = _sc_aggregate(feature, src, dst)
  degT = deg2.reshape(_NC * _NS, _N).T
  return _tc_combine(feature, agg2, degT, W, b)


# speedup vs baseline: 11.2197x; 1.0052x over previous
import functools

import jax
import jax.numpy as jnp
from jax import lax
from jax.experimental import pallas as pl
from jax.experimental.pallas import tpu as pltpu
from jax.experimental.pallas import tpu_sc as plsc

_N = 10000
_D = 128
_E = 320000

_NC = 2
_NS = 16
_NW = _NC * _NS
_EPW = _E // _NW
_B = 50
_NB = _EPW // _B
_CB = 40
_NCH = _NB // _CB
_CE = _CB * _B
_RCH = 16
_RPT = 624


def _sc_aggregate(feature, src, dst):
  mesh = plsc.VectorSubcoreMesh(core_axis_name="c", subcore_axis_name="s")

  @functools.partial(
      pl.kernel,
      out_type=(
          jax.ShapeDtypeStruct((_NC, _N, _D), jnp.float32),
          jax.ShapeDtypeStruct((_NC, _NS, _N), jnp.float32),
      ),
      mesh=mesh,
      compiler_params=pltpu.CompilerParams(needs_layout_passes=False),
      scratch_types=(
          pltpu.VMEM((_CB, _B), jnp.int32),
          pltpu.VMEM((_CB, _B), jnp.int32),
          pltpu.VMEM((3, _B, _D), jnp.float32),
          pltpu.VMEM((_N,), jnp.float32),
          pltpu.VMEM((_RCH, _D), jnp.float32),
          pltpu.VMEM_SHARED((_N, _D), jnp.float32),
          pltpu.SemaphoreType.DMA,
          pltpu.SemaphoreType.DMA,
          pltpu.SemaphoreType.DMA,
          pltpu.SemaphoreType.DMA,
          pltpu.SemaphoreType.DMA,
          pltpu.SemaphoreType.DMA,
      ),
  )
  def k(feature_hbm, src_hbm, dst_hbm, agg_out, deg_out,
        src_v, dst_v, rows_v, hist_v, zrow_v, agg_sh,
        gsem0, gsem1, gsem2, ssem0, ssem1, ssem2):
    cid = lax.axis_index("c")
    sid = lax.axis_index("s")
    wid = sid * _NC + cid
    rbase = sid * _RPT
    nch = jnp.where(sid == _NS - 1, 40, _RPT // _RCH)

    z16 = jnp.zeros((16,), jnp.float32)
    o16 = jnp.ones((16,), jnp.float32)

    @pl.loop(0, _RCH)
    def _zr(r):
      @pl.loop(0, _D // 16)
      def _zc(c):
        zrow_v[r, pl.ds(c * 16, 16)] = z16

    @pl.loop(0, _N // 16)
    def _zh(r):
      hist_v[pl.ds(r * 16, 16)] = z16

    @pl.loop(0, nch)
    def _zi(j):
      pltpu.sync_copy(zrow_v, agg_sh.at[pl.ds(rbase + j * _RCH, _RCH)])

    plsc.subcore_barrier()

    gsems = (gsem0, gsem1, gsem2)
    ssems = (ssem0, ssem1, ssem2)
    tailmask = lax.iota(jnp.int32, 16) >= (16 - _B % 16)

    def _start_gather(i, buf):
      pltpu.async_copy(feature_hbm.at[src_v.at[i]], rows_v.at[buf], gsems[buf])

    def _wait_gather(i, buf):
      pltpu.make_async_copy(feature_hbm.at[src_v.at[i]], rows_v.at[buf],
                            gsems[buf]).wait()

    def _start_scatter(i, buf):
      pltpu.async_copy(rows_v.at[buf], agg_sh.at[dst_v.at[i]], ssems[buf],
                       add=True)

    def _wait_scatter(i, buf):
      pltpu.make_async_copy(rows_v.at[buf], agg_sh.at[dst_v.at[i]],
                            ssems[buf]).wait()

    def _hist(i):
      @pl.loop(0, _B // 16)
      def _dh(g):
        plsc.addupdate_scatter(hist_v, [dst_v[i, pl.ds(g * 16, 16)]], o16)

      plsc.addupdate_scatter(hist_v, [dst_v[i, pl.ds(_B - 16, 16)]], o16,
                             mask=tailmask)

    @pl.loop(0, _NCH)
    def _ch(c):
      pltpu.sync_copy(src_hbm.at[wid, pl.ds(c * _CB, _CB)], src_v)
      pltpu.sync_copy(dst_hbm.at[wid, pl.ds(c * _CB, _CB)], dst_v)
      for j in range(3):
        _start_gather(j, j)

      @pl.loop(0, _CB - 6, step=3)
      def _eb(i):
        for j in range(3):
          _wait_gather(i + j, j)
          _start_scatter(i + j, j)
          _hist(i + j)

        for j in range(3):
          _wait_scatter(i + j, j)
          _start_gather(i + j + 3, j)

      _wait_gather(_CB - 4, 0)
      _start_scatter(_CB - 4, 0)
      _hist(_CB - 4)
      _wait_scatter(_CB - 4, 0)
      _start_gather(_CB - 1, 0)
      for k in (_CB - 3, _CB - 2, _CB - 1):
        _wait_gather(k, k % 3)
        _start_scatter(k, k % 3)
        _hist(k)
      for k in (_CB - 3, _CB - 2, _CB - 1):
        _wait_scatter(k, k % 3)

    plsc.subcore_barrier()

    @pl.loop(0, nch)
    def _wo(j):
      r0 = rbase + j * _RCH
      pltpu.sync_copy(agg_sh.at[pl.ds(r0, _RCH)], zrow_v)
      pltpu.sync_copy(zrow_v, agg_out.at[cid, pl.ds(r0, _RCH)])

    pltpu.sync_copy(hist_v, deg_out.at[cid, sid])

  return k(feature, src.reshape(_NW, _NB, _B), dst.reshape(_NW, _NB, _B))


_R = 1000


def _tc_combine(feature, agg2, degT, W, b):
  def body(f_ref, a_ref, d_ref, w_ref, b_ref, o_ref):
    f = f_ref[...]
    a = a_ref[0] + a_ref[1]
    d = jnp.sum(d_ref[...], axis=1)
    deg = jnp.maximum(d, 1.0)[:, None]
    neigh = a / deg
    w = w_ref[...]
    acc = jnp.dot(f, w[:_D], preferred_element_type=jnp.float32)
    acc = acc + jnp.dot(neigh, w[_D:], preferred_element_type=jnp.float32)
    o_ref[...] = jnp.maximum(acc + b_ref[...], 0.0)

  return pl.pallas_call(
      body,
      grid=(_N // _R,),
      in_specs=[
          pl.BlockSpec((_R, _D), lambda i: (i, 0)),
          pl.BlockSpec((_NC, _R, _D), lambda i: (0, i, 0)),
          pl.BlockSpec((_R, _NC * _NS), lambda i: (i, 0)),
          pl.BlockSpec((2 * _D, _D), lambda i: (0, 0)),
          pl.BlockSpec((1, _D), lambda i: (0, 0)),
      ],
      out_specs=pl.BlockSpec((_R, _D), lambda i: (i, 0)),
      out_shape=jax.ShapeDtypeStruct((_N, _D), jnp.float32),
  )(feature, agg2, degT, W, b.reshape(1, _D))


def kernel(feature, edge_index, W, b):
  src = edge_index[0]
  dst = edge_index[1]
  agg2, deg2 = _sc_aggregate(feature, src, dst)
  degT = deg2.reshape(_NC * _NS, _N).T
  return _tc_combine(feature, agg2, degT, W, b)


# speedup vs baseline: 11.2270x; 1.0006x over previous
import functools

import jax
import jax.numpy as jnp
from jax import lax
from jax.experimental import pallas as pl
from jax.experimental.pallas import tpu as pltpu
from jax.experimental.pallas import tpu_sc as plsc

_N = 10000
_D = 128
_E = 320000

_NC = 2
_NS = 16
_NW = _NC * _NS
_EPW = _E // _NW
_B = 50
_NB = _EPW // _B
_CB = 40
_NCH = _NB // _CB
_CE = _CB * _B
_RCH = 16
_RPT = 624


def _sc_aggregate(feature, src, dst):
  mesh = plsc.VectorSubcoreMesh(core_axis_name="c", subcore_axis_name="s")

  @functools.partial(
      pl.kernel,
      out_type=(
          jax.ShapeDtypeStruct((_NC, _N, _D), jnp.float32),
          jax.ShapeDtypeStruct((_NC, _NS, _N), jnp.float32),
      ),
      mesh=mesh,
      compiler_params=pltpu.CompilerParams(needs_layout_passes=False),
      scratch_types=(
          pltpu.VMEM((_CB, _B), jnp.int32),
          pltpu.VMEM((_CB, _B), jnp.int32),
          pltpu.VMEM((3, _B, _D), jnp.float32),
          pltpu.VMEM((_N,), jnp.float32),
          pltpu.VMEM((_RCH, _D), jnp.float32),
          pltpu.VMEM_SHARED((_N, _D), jnp.float32),
          pltpu.SemaphoreType.DMA,
          pltpu.SemaphoreType.DMA,
          pltpu.SemaphoreType.DMA,
          pltpu.SemaphoreType.DMA,
          pltpu.SemaphoreType.DMA,
          pltpu.SemaphoreType.DMA,
      ),
  )
  def k(feature_hbm, src_hbm, dst_hbm, agg_out, deg_out,
        src_v, dst_v, rows_v, hist_v, zrow_v, agg_sh,
        gsem0, gsem1, gsem2, ssem0, ssem1, ssem2):
    cid = lax.axis_index("c")
    sid = lax.axis_index("s")
    wid = sid * _NC + cid
    rbase = sid * _RPT
    nch = jnp.where(sid == _NS - 1, 40, _RPT // _RCH)

    z16 = jnp.zeros((16,), jnp.float32)
    o16 = jnp.ones((16,), jnp.float32)

    @pl.loop(0, _RCH)
    def _zr(r):
      @pl.loop(0, _D // 16)
      def _zc(c):
        zrow_v[r, pl.ds(c * 16, 16)] = z16

    @pl.loop(0, _N // 16)
    def _zh(r):
      hist_v[pl.ds(r * 16, 16)] = z16

    @pl.loop(0, nch)
    def _zi(j):
      pltpu.sync_copy(zrow_v, agg_sh.at[pl.ds(rbase + j * _RCH, _RCH)])

    plsc.subcore_barrier()

    gsems = (gsem0, gsem1, gsem2)
    ssems = (ssem0, ssem1, ssem2)
    tailmask = lax.iota(jnp.int32, 16) >= (16 - _B % 16)

    def _start_gather(i, buf):
      pltpu.async_copy(feature_hbm.at[src_v.at[i]], rows_v.at[buf], gsems[buf])

    def _wait_gather(i, buf):
      pltpu.make_async_copy(feature_hbm.at[src_v.at[i]], rows_v.at[buf],
                            gsems[buf]).wait()

    def _start_scatter(i, buf):
      pltpu.async_copy(rows_v.at[buf], agg_sh.at[dst_v.at[i]], ssems[buf],
                       add=True)

    def _wait_scatter(i, buf):
      pltpu.make_async_copy(rows_v.at[buf], agg_sh.at[dst_v.at[i]],
                            ssems[buf]).wait()

    def _hist(i):
      @pl.loop(0, _B // 16)
      def _dh(g):
        plsc.addupdate_scatter(hist_v, [dst_v[i, pl.ds(g * 16, 16)]], o16)

      plsc.addupdate_scatter(hist_v, [dst_v[i, pl.ds(_B - 16, 16)]], o16,
                             mask=tailmask)

    @pl.loop(0, _NCH)
    def _ch(c):
      pltpu.sync_copy(src_hbm.at[wid, pl.ds(c * _CB, _CB)], src_v)
      pltpu.sync_copy(dst_hbm.at[wid, pl.ds(c * _CB, _CB)], dst_v)
      for j in range(3):
        _start_gather(j, j)

      @pl.loop(0, _CB - 6, step=3, unroll=2)
      def _eb(i):
        for j in range(3):
          _wait_gather(i + j, j)
          _start_scatter(i + j, j)
          _hist(i + j)

        for j in range(3):
          _wait_scatter(i + j, j)
          _start_gather(i + j + 3, j)

      _wait_gather(_CB - 4, 0)
      _start_scatter(_CB - 4, 0)
      _hist(_CB - 4)
      _wait_scatter(_CB - 4, 0)
      _start_gather(_CB - 1, 0)
      for k in (_CB - 3, _CB - 2, _CB - 1):
        _wait_gather(k, k % 3)
        _start_scatter(k, k % 3)
        _hist(k)
      for k in (_CB - 3, _CB - 2, _CB - 1):
        _wait_scatter(k, k % 3)

    plsc.subcore_barrier()

    @pl.loop(0, nch)
    def _wo(j):
      r0 = rbase + j * _RCH
      pltpu.sync_copy(agg_sh.at[pl.ds(r0, _RCH)], zrow_v)
      pltpu.sync_copy(zrow_v, agg_out.at[cid, pl.ds(r0, _RCH)])

    pltpu.sync_copy(hist_v, deg_out.at[cid, sid])

  return k(feature, src.reshape(_NW, _NB, _B), dst.reshape(_NW, _NB, _B))


_R = 1000


def _tc_combine(feature, agg2, degT, W, b):
  def body(f_ref, a_ref, d_ref, w_ref, b_ref, o_ref):
    f = f_ref[...]
    a = a_ref[0] + a_ref[1]
    d = jnp.sum(d_ref[...], axis=1)
    deg = jnp.maximum(d, 1.0)[:, None]
    neigh = a / deg
    w = w_ref[...]
    acc = jnp.dot(f, w[:_D], preferred_element_type=jnp.float32)
    acc = acc + jnp.dot(neigh, w[_D:], preferred_element_type=jnp.float32)
    o_ref[...] = jnp.maximum(acc + b_ref[...], 0.0)

  return pl.pallas_call(
      body,
      grid=(_N // _R,),
      in_specs=[
          pl.BlockSpec((_R, _D), lambda i: (i, 0)),
          pl.BlockSpec((_NC, _R, _D), lambda i: (0, i, 0)),
          pl.BlockSpec((_R, _NC * _NS), lambda i: (i, 0)),
          pl.BlockSpec((2 * _D, _D), lambda i: (0, 0)),
          pl.BlockSpec((1, _D), lambda i: (0, 0)),
      ],
      out_specs=pl.BlockSpec((_R, _D), lambda i: (i, 0)),
      out_shape=jax.ShapeDtypeStruct((_N, _D), jnp.float32),
  )(feature, agg2, degT, W, b.reshape(1, _D))


def kernel(feature, edge_index, W, b):
  src = edge_index[0]
  dst = edge_index[1]
  agg2, deg2 = _sc_aggregate(feature, src, dst)
  degT = deg2.reshape(_NC * _NS, _N).T
  return _tc_combine(feature, agg2, degT, W, b)
